# baseline (speedup 1.0000x reference)
import functools
import math

import jax
import jax.numpy as jnp
from jax import lax
from jax.experimental import pallas as pl
from jax.experimental.pallas import tpu as pltpu
from jax.experimental.pallas import tpu_sc as plsc

EMBED_DIM = 1024
HALF_DIM = EMBED_DIM // 2
PADDING_IDX = 1

NUM_CORES = 2
NUM_SUBCORES = 16
NUM_WORKERS = NUM_CORES * NUM_SUBCORES

TBLK = 128
CHUNK = 32
LANES = 16


def _table_body(o_ref, srf, crf, *, seq_len):
    i = pl.program_id(0)
    k = lax.broadcasted_iota(jnp.int32, (1, HALF_DIM), 1).astype(jnp.float32)
    inv_freq = jnp.exp(k * (-math.log(10000.0) / (HALF_DIM - 1)))

    def pack(s, c):
        s16 = lax.bitcast_convert_type(s.astype(jnp.bfloat16), jnp.uint16)
        c16 = lax.bitcast_convert_type(c.astype(jnp.bfloat16), jnp.uint16)
        return (s16.astype(jnp.int32) | (c16.astype(jnp.int32) << 16))

    @pl.when(i == 0)
    def _():
        r = lax.broadcasted_iota(jnp.int32, (TBLK, 1), 0).astype(jnp.float32)
        ang = (r + float(PADDING_IDX + 1)) * inv_freq
        s, c = jnp.sin(ang), jnp.cos(ang)
        srf[...] = s
        crf[...] = c
        o_ref[...] = pack(s, c)

    @pl.when(i > 0)
    def _():
        ang_b = (i * TBLK).astype(jnp.float32) * inv_freq
        sb, cb = jnp.sin(ang_b), jnp.cos(ang_b)
        s0, c0 = srf[...], crf[...]
        o_ref[...] = pack(s0 * cb + c0 * sb, c0 * cb - s0 * sb)


def _build_table(seq_len):
    return pl.pallas_call(
        functools.partial(_table_body, seq_len=seq_len),
        out_shape=jax.ShapeDtypeStruct((seq_len, HALF_DIM), jnp.int32),
        grid=(seq_len // TBLK,),
        out_specs=pl.BlockSpec((TBLK, HALF_DIM), lambda i: (i, 0)),
        scratch_shapes=[
            pltpu.VMEM((TBLK, HALF_DIM), jnp.float32),
            pltpu.VMEM((TBLK, HALF_DIM), jnp.float32),
        ],
    )()


def _splat16(x, r):
    return lax.gather(
        x,
        jnp.full((LANES, 1), r, jnp.int32),
        lax.GatherDimensionNumbers(
            offset_dims=(), collapsed_slice_dims=(0,), start_index_map=(0,)
        ),
        slice_sizes=(1,),
        mode=lax.GatherScatterMode.PROMISE_IN_BOUNDS,
    )


def _gather_body(tok_hbm, table_hbm, out_hbm, tok_v, gbuf, sbuf, sg, ss, *,
                 bsz, seq_len, jcols):
    wid = lax.axis_index("s") * NUM_CORES + lax.axis_index("c")
    jbase = wid * jcols

    for b in range(bsz):
        pltpu.sync_copy(tok_hbm.at[b].at[pl.ds(jbase, jcols)], tok_v.at[b])

    nchunks = jcols // CHUNK
    gcp = [None, None]

    def start_gather(c):
        slot = c % 2
        gcp[slot] = pltpu.make_async_copy(
            table_hbm.at[pl.ds(jbase + c * CHUNK, CHUNK)], gbuf.at[slot],
            sg.at[slot])
        gcp[slot].start()

    scp = [None, None]
    start_gather(0)
    step = 0
    for c in range(nchunks):
        slot = c % 2
        gcp[slot].wait()
        if c + 1 < nchunks:
            start_gather(c + 1)
        for b in range(bsz):
            sslot = step % 2
            if step >= 2:
                scp[sslot].wait()
            for h in range(CHUNK // LANES):
                tok16 = tok_v[b, pl.ds(c * CHUNK + h * LANES, LANES)]
                fvec = jnp.where(tok16 == PADDING_IDX, 0.0, 1.0)

                @plsc.parallel_loop(0, LANES)
                def _row(r, fvec=fvec, slot=slot, sslot=sslot, h=h):
                    f = _splat16(fvec, r)
                    rr = h * LANES + r

                    @plsc.parallel_loop(0, HALF_DIM, step=LANES, unroll=8)
                    def _col(o):
                        x = gbuf[slot, rr, pl.ds(o, LANES)]
                        s = lax.bitcast_convert_type(x << 16, jnp.float32)
                        c = lax.bitcast_convert_type(x & jnp.int32(-65536),
                                                     jnp.float32)
                        sbuf[sslot, rr, pl.ds(o, LANES)] = s * f
                        sbuf[sslot, rr, pl.ds(HALF_DIM + o, LANES)] = c * f

            scp[sslot] = pltpu.make_async_copy(
                sbuf.at[sslot],
                out_hbm.at[pl.ds(b * seq_len + jbase + c * CHUNK, CHUNK)],
                ss.at[sslot])
            scp[sslot].start()
            step += 1
    for sslot in range(2):
        scp[sslot].wait()


def _gather(tokens, table, bsz, seq_len):
    jcols = seq_len // NUM_WORKERS
    total = bsz * seq_len
    mesh = plsc.VectorSubcoreMesh(
        core_axis_name="c",
        subcore_axis_name="s",
        num_cores=NUM_CORES,
        num_subcores=NUM_SUBCORES,
    )
    body = functools.partial(
        _gather_body, bsz=bsz, seq_len=seq_len, jcols=jcols
    )
    return pl.kernel(
        body,
        out_type=jax.ShapeDtypeStruct((total, EMBED_DIM), jnp.float32),
        mesh=mesh,
        scratch_types=[
            pltpu.VMEM((bsz, jcols), jnp.int32),
            pltpu.VMEM((2, CHUNK, HALF_DIM), jnp.int32),
            pltpu.VMEM((2, CHUNK, EMBED_DIM), jnp.float32),
            pltpu.SemaphoreType.DMA((2,)),
            pltpu.SemaphoreType.DMA((2,)),
        ],
    )(tokens, table)


def kernel(tokens):
    bsz, seq_len = tokens.shape
    table = _build_table(seq_len)
    out = _gather(tokens, table, bsz, seq_len)
    return out.reshape(bsz, seq_len, EMBED_DIM)


# speedup vs baseline: 1.0209x; 1.0209x over previous
import functools
import math

import jax
import jax.numpy as jnp
from jax import lax
from jax.experimental import pallas as pl
from jax.experimental.pallas import tpu as pltpu
from jax.experimental.pallas import tpu_sc as plsc

EMBED_DIM = 1024
HALF_DIM = EMBED_DIM // 2
PADDING_IDX = 1

NUM_CORES = 2
NUM_SUBCORES = 16
NUM_WORKERS = NUM_CORES * NUM_SUBCORES

TBLK = 128
CHUNK = 16
LANES = 16


def _table_body(o_ref, srf, crf, *, seq_len):
    i = pl.program_id(0)
    k = lax.broadcasted_iota(jnp.int32, (1, HALF_DIM), 1).astype(jnp.float32)
    inv_freq = jnp.exp(k * (-math.log(10000.0) / (HALF_DIM - 1)))

    def pack(s, c):
        s16 = lax.bitcast_convert_type(s.astype(jnp.bfloat16), jnp.uint16)
        c16 = lax.bitcast_convert_type(c.astype(jnp.bfloat16), jnp.uint16)
        return (s16.astype(jnp.int32) | (c16.astype(jnp.int32) << 16))

    @pl.when(i == 0)
    def _():
        r = lax.broadcasted_iota(jnp.int32, (TBLK, 1), 0).astype(jnp.float32)
        ang = (r + float(PADDING_IDX + 1)) * inv_freq
        s, c = jnp.sin(ang), jnp.cos(ang)
        srf[...] = s
        crf[...] = c
        o_ref[...] = pack(s, c)

    @pl.when(i > 0)
    def _():
        ang_b = (i * TBLK).astype(jnp.float32) * inv_freq
        sb, cb = jnp.sin(ang_b), jnp.cos(ang_b)
        s0, c0 = srf[...], crf[...]
        o_ref[...] = pack(s0 * cb + c0 * sb, c0 * cb - s0 * sb)


def _build_table(seq_len):
    return pl.pallas_call(
        functools.partial(_table_body, seq_len=seq_len),
        out_shape=jax.ShapeDtypeStruct((seq_len, HALF_DIM), jnp.int32),
        grid=(seq_len // TBLK,),
        out_specs=pl.BlockSpec((TBLK, HALF_DIM), lambda i: (i, 0)),
        scratch_shapes=[
            pltpu.VMEM((TBLK, HALF_DIM), jnp.float32),
            pltpu.VMEM((TBLK, HALF_DIM), jnp.float32),
        ],
    )()


def _splat16(x, r):
    return lax.gather(
        x,
        jnp.full((LANES, 1), r, jnp.int32),
        lax.GatherDimensionNumbers(
            offset_dims=(), collapsed_slice_dims=(0,), start_index_map=(0,)
        ),
        slice_sizes=(1,),
        mode=lax.GatherScatterMode.PROMISE_IN_BOUNDS,
    )


def _gather_body(tok_hbm, table_hbm, out_hbm, tok_v, gbuf, sbuf, sg, ss, *,
                 bsz, seq_len, jcols):
    wid = lax.axis_index("s") * NUM_CORES + lax.axis_index("c")
    jbase = wid * jcols

    for b in range(bsz):
        pltpu.sync_copy(tok_hbm.at[b].at[pl.ds(jbase, jcols)], tok_v.at[b])

    nchunks = jcols // CHUNK
    gcp = [None, None]

    def start_gather(c):
        slot = c % 2
        gcp[slot] = pltpu.make_async_copy(
            table_hbm.at[pl.ds(jbase + c * CHUNK, CHUNK)], gbuf.at[slot],
            sg.at[slot])
        gcp[slot].start()

    scp = [None, None, None]
    start_gather(0)
    step = 0
    for c in range(nchunks):
        slot = c % 2
        gcp[slot].wait()
        if c + 1 < nchunks:
            start_gather(c + 1)
        for b in range(bsz):
            sslot = step % 3
            if step >= 3:
                scp[sslot].wait()
            tok16 = tok_v[b, pl.ds(c * CHUNK, CHUNK)]
            fvec = jnp.where(tok16 == PADDING_IDX, 0.0, 1.0)

            @plsc.parallel_loop(0, CHUNK)
            def _row(r, fvec=fvec, slot=slot, sslot=sslot):
                f = _splat16(fvec, r)

                @plsc.parallel_loop(0, HALF_DIM, step=LANES, unroll=8)
                def _col(o):
                    x = gbuf[slot, r, pl.ds(o, LANES)]
                    s = lax.bitcast_convert_type(x << 16, jnp.float32)
                    c = lax.bitcast_convert_type(x & jnp.int32(-65536),
                                                 jnp.float32)
                    sbuf[sslot, r, pl.ds(o, LANES)] = s * f
                    sbuf[sslot, r, pl.ds(HALF_DIM + o, LANES)] = c * f

            scp[sslot] = pltpu.make_async_copy(
                sbuf.at[sslot],
                out_hbm.at[pl.ds(b * seq_len + jbase + c * CHUNK, CHUNK)],
                ss.at[sslot])
            scp[sslot].start()
            step += 1
    for sslot in range(3):
        scp[sslot].wait()


def _gather(tokens, table, bsz, seq_len):
    jcols = seq_len // NUM_WORKERS
    total = bsz * seq_len
    mesh = plsc.VectorSubcoreMesh(
        core_axis_name="c",
        subcore_axis_name="s",
        num_cores=NUM_CORES,
        num_subcores=NUM_SUBCORES,
    )
    body = functools.partial(
        _gather_body, bsz=bsz, seq_len=seq_len, jcols=jcols
    )
    return pl.kernel(
        body,
        out_type=jax.ShapeDtypeStruct((total, EMBED_DIM), jnp.float32),
        mesh=mesh,
        scratch_types=[
            pltpu.VMEM((bsz, jcols), jnp.int32),
            pltpu.VMEM((2, CHUNK, HALF_DIM), jnp.int32),
            pltpu.VMEM((3, CHUNK, EMBED_DIM), jnp.float32),
            pltpu.SemaphoreType.DMA((2,)),
            pltpu.SemaphoreType.DMA((3,)),
        ],
    )(tokens, table)


def kernel(tokens):
    bsz, seq_len = tokens.shape
    table = _build_table(seq_len)
    out = _gather(tokens, table, bsz, seq_len)
    return out.reshape(bsz, seq_len, EMBED_DIM)


# speedup vs baseline: 1.0252x; 1.0042x over previous
import functools
import math

import jax
import jax.numpy as jnp
from jax import lax
from jax.experimental import pallas as pl
from jax.experimental.pallas import tpu as pltpu
from jax.experimental.pallas import tpu_sc as plsc

EMBED_DIM = 1024
HALF_DIM = EMBED_DIM // 2
PADDING_IDX = 1

NUM_CORES = 2
NUM_SUBCORES = 16
NUM_WORKERS = NUM_CORES * NUM_SUBCORES

TBLK = 128
CHUNK = 16
LANES = 16


def _table_body(o_ref, srf, crf, *, seq_len):
    i = pl.program_id(0)
    k = lax.broadcasted_iota(jnp.int32, (1, HALF_DIM), 1).astype(jnp.float32)
    inv_freq = jnp.exp(k * (-math.log(10000.0) / (HALF_DIM - 1)))

    def pack(s, c):
        s16 = lax.bitcast_convert_type(s.astype(jnp.bfloat16), jnp.uint16)
        c16 = lax.bitcast_convert_type(c.astype(jnp.bfloat16), jnp.uint16)
        return (s16.astype(jnp.int32) | (c16.astype(jnp.int32) << 16))

    @pl.when(i == 0)
    def _():
        r = lax.broadcasted_iota(jnp.int32, (TBLK, 1), 0).astype(jnp.float32)
        ang = (r + float(PADDING_IDX + 1)) * inv_freq
        s, c = jnp.sin(ang), jnp.cos(ang)
        srf[...] = s
        crf[...] = c
        o_ref[...] = pack(s, c)

    @pl.when(i > 0)
    def _():
        ang_b = (i * TBLK).astype(jnp.float32) * inv_freq
        sb, cb = jnp.sin(ang_b), jnp.cos(ang_b)
        s0, c0 = srf[...], crf[...]
        o_ref[...] = pack(s0 * cb + c0 * sb, c0 * cb - s0 * sb)


def _build_table(seq_len):
    return pl.pallas_call(
        functools.partial(_table_body, seq_len=seq_len),
        out_shape=jax.ShapeDtypeStruct((seq_len, HALF_DIM), jnp.int32),
        grid=(seq_len // TBLK,),
        out_specs=pl.BlockSpec((TBLK, HALF_DIM), lambda i: (i, 0)),
        scratch_shapes=[
            pltpu.VMEM((TBLK, HALF_DIM), jnp.float32),
            pltpu.VMEM((TBLK, HALF_DIM), jnp.float32),
        ],
    )()


def _splat16(x, r):
    return lax.gather(
        x,
        jnp.full((LANES, 1), r, jnp.int32),
        lax.GatherDimensionNumbers(
            offset_dims=(), collapsed_slice_dims=(0,), start_index_map=(0,)
        ),
        slice_sizes=(1,),
        mode=lax.GatherScatterMode.PROMISE_IN_BOUNDS,
    )


def _gather_body(tok_hbm, table_hbm, out_hbm, tok_v, gbuf, sbuf, sg, ss, *,
                 bsz, seq_len, jcols):
    wid = lax.axis_index("s") * NUM_CORES + lax.axis_index("c")
    jbase = wid * jcols

    for b in range(bsz):
        pltpu.sync_copy(tok_hbm.at[b].at[pl.ds(jbase, jcols)], tok_v.at[b])

    nchunks = jcols // CHUNK
    gcp = [None, None]

    def start_gather(c):
        slot = c % 2
        gcp[slot] = pltpu.make_async_copy(
            table_hbm.at[pl.ds(jbase + c * CHUNK, CHUNK)], gbuf.at[slot],
            sg.at[slot])
        gcp[slot].start()

    scp = [None, None]
    start_gather(0)
    step = 0
    for c in range(nchunks):
        slot = c % 2
        gcp[slot].wait()
        if c + 1 < nchunks:
            start_gather(c + 1)
        for b in range(bsz):
            sslot = step % 2
            if step >= 2:
                scp[sslot].wait()
            tok16 = tok_v[b, pl.ds(c * CHUNK, CHUNK)]
            fvec = jnp.where(tok16 == PADDING_IDX, 0.0, 1.0)

            @plsc.parallel_loop(0, CHUNK)
            def _row(r, fvec=fvec, slot=slot, sslot=sslot):
                f = _splat16(fvec, r)

                @plsc.parallel_loop(0, HALF_DIM, step=LANES, unroll=8)
                def _col(o):
                    x = gbuf[slot, r, pl.ds(o, LANES)]
                    s = lax.bitcast_convert_type(x << 16, jnp.float32)
                    c = lax.bitcast_convert_type(x & jnp.int32(-65536),
                                                 jnp.float32)
                    sbuf[sslot, r, pl.ds(o, LANES)] = s * f
                    sbuf[sslot, r, pl.ds(HALF_DIM + o, LANES)] = c * f

            scp[sslot] = pltpu.make_async_copy(
                sbuf.at[sslot],
                out_hbm.at[pl.ds(b * seq_len + jbase + c * CHUNK, CHUNK)],
                ss.at[sslot])
            scp[sslot].start()
            step += 1
    for sslot in range(2):
        scp[sslot].wait()


def _gather(tokens, table, bsz, seq_len):
    jcols = seq_len // NUM_WORKERS
    total = bsz * seq_len
    mesh = plsc.VectorSubcoreMesh(
        core_axis_name="c",
        subcore_axis_name="s",
        num_cores=NUM_CORES,
        num_subcores=NUM_SUBCORES,
    )
    body = functools.partial(
        _gather_body, bsz=bsz, seq_len=seq_len, jcols=jcols
    )
    return pl.kernel(
        body,
        out_type=jax.ShapeDtypeStruct((total, EMBED_DIM), jnp.float32),
        mesh=mesh,
        scratch_types=[
            pltpu.VMEM((bsz, jcols), jnp.int32),
            pltpu.VMEM((2, CHUNK, HALF_DIM), jnp.int32),
            pltpu.VMEM((2, CHUNK, EMBED_DIM), jnp.float32),
            pltpu.SemaphoreType.DMA((2,)),
            pltpu.SemaphoreType.DMA((2,)),
        ],
    )(tokens, table)


def kernel(tokens):
    bsz, seq_len = tokens.shape
    table = _build_table(seq_len)
    out = _gather(tokens, table, bsz, seq_len)
    return out.reshape(bsz, seq_len, EMBED_DIM)
